# baseline (speedup 1.0000x reference)
import functools

import jax
import jax.numpy as jnp
from jax import lax
from jax.experimental import pallas as pl
from jax.experimental.pallas import tpu as pltpu
from jax.experimental.pallas import tpu_sc as plsc

_N = 10000
_E = 320000
_D = 128
_H = 16
_G = 128
_OUT = 10
_NP = 10240
_R = 1024
_NBLK = _NP // _R

_SC_PARAMS = pltpu.CompilerParams(needs_layout_passes=False)


def _sc_mesh():
    return plsc.VectorSubcoreMesh(core_axis_name="c", subcore_axis_name="s",
                                  num_cores=2, num_subcores=16)


def _rsqrt_newton(x):
    i = lax.bitcast_convert_type(x, jnp.int32)
    i = jnp.int32(0x5F3759DF) - lax.shift_right_arithmetic(i, 1)
    y = lax.bitcast_convert_type(i, jnp.float32)
    for _ in range(3):
        y = y * (1.5 - 0.5 * x * y * y)
    return y


def _att_body(x_ref, wa_ref, ba_ref, w1_ref, o_ref):
    lt = lax.dot_general(wa_ref[...], x_ref[...], (((1,), (1,)), ((), ())),
                         preferred_element_type=jnp.float32)
    lt = lt + ba_ref[...]
    m = jnp.max(lt, axis=0, keepdims=True)
    e = jnp.exp(lt - m)
    aT = e / jnp.sum(e, axis=0, keepdims=True)
    o_ref[...] = lax.dot_general(w1_ref[...], aT, (((1,), (0,)), ((), ())),
                                 preferred_element_type=jnp.float32)


def _attention(x_p, W_att, b_att, W1):
    return pl.pallas_call(
        _att_body,
        grid=(_NBLK,),
        in_specs=[
            pl.BlockSpec((_R, _D), lambda i: (i, 0)),
            pl.BlockSpec((_H, _D), lambda i: (0, 0)),
            pl.BlockSpec((_H, 1), lambda i: (0, 0)),
            pl.BlockSpec((_H, _H), lambda i: (0, 0)),
        ],
        out_specs=pl.BlockSpec((_H, _R), lambda i: (0, i)),
        out_shape=jax.ShapeDtypeStruct((_H, _NP), jnp.float32),
    )(x_p, W_att, b_att.reshape(_H, 1), W1)


def _norm_sc(src, dst, ew):
    deg_ch, deg_nch = 4000, 5
    nm_ch, nm_nch = 2000, 5

    @functools.partial(
        pl.kernel,
        out_type=[
            jax.ShapeDtypeStruct((_NP,), jnp.float32),
            jax.ShapeDtypeStruct((_E,), jnp.float32),
        ],
        mesh=_sc_mesh(),
        scratch_types=[
            pltpu.VMEM((_NP,), jnp.float32),
            pltpu.VMEM((_NP,), jnp.float32),
            pltpu.VMEM((deg_ch,), jnp.int32),
            pltpu.VMEM((nm_ch,), jnp.int32),
            pltpu.VMEM((deg_ch,), jnp.float32),
            pltpu.VMEM((nm_ch,), jnp.float32),
            pltpu.VMEM((640,), jnp.float32),
            pltpu.VMEM((640,), jnp.float32),
            pltpu.VMEM_SHARED((16 * _NP,), jnp.float32),
            pltpu.VMEM_SHARED((_NP,), jnp.float32),
        ],
        compiler_params=_SC_PARAMS,
    )
    def k(src_hbm, dst_hbm, ew_hbm, selfw_hbm, norm_hbm,
          deg_v, dinv_v, ib1, ib2, fb1, nb, a640, t640, sm_degs, sm_dinv):
        cid = lax.axis_index("c")
        sid = lax.axis_index("s")
        wid = cid * 16 + sid

        def zero_body(i):
            deg_v[pl.ds(i * 16, 16)] = jnp.zeros((16,), jnp.float32)
        plsc.parallel_loop(0, _NP // 16, 1, unroll=8)(zero_body)

        def deg_chunk(i, _):
            b = sid * 20000 + i * deg_ch
            pltpu.sync_copy(dst_hbm.at[pl.ds(b, deg_ch)], ib1)
            pltpu.sync_copy(ew_hbm.at[pl.ds(b, deg_ch)], fb1)

            def grp(j):
                d16 = ib1[pl.ds(j * 16, 16)]
                w16 = fb1[pl.ds(j * 16, 16)]
                plsc.addupdate_scatter(deg_v, [d16], w16)
            plsc.parallel_loop(0, deg_ch // 16, 1, unroll=10)(grp)
            return 0
        lax.fori_loop(0, deg_nch, deg_chunk, 0)

        pltpu.sync_copy(deg_v, sm_degs.at[pl.ds(sid * _NP, _NP)])
        plsc.subcore_barrier()

        pltpu.sync_copy(sm_degs.at[pl.ds(sid * 640, 640)], a640)
        for t in range(1, 16):
            pltpu.sync_copy(sm_degs.at[pl.ds(t * _NP + sid * 640, 640)], t640)

            def add_body(kk, _t=t):
                a640[pl.ds(kk * 16, 16)] = (a640[pl.ds(kk * 16, 16)] +
                                            t640[pl.ds(kk * 16, 16)])
            plsc.parallel_loop(0, 40, 1, unroll=8)(add_body)

        def dinv_body(j):
            d = a640[pl.ds(j * 16, 16)] + 1.0
            y = _rsqrt_newton(d)
            nb[pl.ds(j * 16, 16)] = y
            fb1[pl.ds(1024 + j * 16, 16)] = y * y
        plsc.parallel_loop(0, 40, 1, unroll=8)(dinv_body)
        pltpu.sync_copy(nb.at[pl.ds(0, 640)],
                        sm_dinv.at[pl.ds(sid * 640, 640)])

        @pl.when(cid == 0)
        def _():
            pltpu.sync_copy(fb1.at[pl.ds(1024, 640)],
                            selfw_hbm.at[pl.ds(sid * 640, 640)])
        plsc.subcore_barrier()
        pltpu.sync_copy(sm_dinv, dinv_v)

        def nm_chunk(i, _):
            b = wid * 10000 + i * nm_ch
            pltpu.sync_copy(src_hbm.at[pl.ds(b, nm_ch)], ib2)
            pltpu.sync_copy(dst_hbm.at[pl.ds(b, nm_ch)],
                            ib1.at[pl.ds(0, nm_ch)])
            pltpu.sync_copy(ew_hbm.at[pl.ds(b, nm_ch)],
                            fb1.at[pl.ds(0, nm_ch)])

            def grp(j):
                s16 = ib2[pl.ds(j * 16, 16)]
                d16 = ib1[pl.ds(j * 16, 16)]
                w16 = fb1[pl.ds(j * 16, 16)]
                nv = (plsc.load_gather(dinv_v, [s16]) * w16 *
                      plsc.load_gather(dinv_v, [d16]))
                nb[pl.ds(j * 16, 16)] = nv
            plsc.parallel_loop(0, nm_ch // 16, 1, unroll=10)(grp)
            pltpu.sync_copy(nb, norm_hbm.at[pl.ds(b, nm_ch)])
            return 0
        lax.fori_loop(0, nm_nch, nm_chunk, 0)

    return k(src, dst, ew)


def _make_agg(F, n_fg, n_es, ch, unroll=25):
    FP = F // n_fg
    edges = _E // n_es
    nch = edges // ch
    assert ch % (16 * unroll) == 0 and edges % ch == 0

    @functools.partial(
        pl.kernel,
        out_type=jax.ShapeDtypeStruct((n_es * F * _NP,), jnp.float32),
        mesh=_sc_mesh(),
        scratch_types=(
            [pltpu.VMEM((_NP,), jnp.float32) for _ in range(FP)] +
            [pltpu.VMEM((_NP,), jnp.float32) for _ in range(FP)] +
            [pltpu.VMEM((ch,), jnp.int32) for _ in range(2)] +
            [pltpu.VMEM((ch,), jnp.int32) for _ in range(2)] +
            [pltpu.VMEM((ch,), jnp.float32) for _ in range(2)] +
            [pltpu.SemaphoreType.DMA, pltpu.SemaphoreType.DMA]
        ),
        compiler_params=_SC_PARAMS,
    )
    def k(xwT_hbm, src_hbm, dst_hbm, norm_hbm, out_hbm, *scratch):
        tbl = scratch[:FP]
        acc = scratch[FP:2 * FP]
        ibs = scratch[2 * FP:2 * FP + 2]
        ids = scratch[2 * FP + 2:2 * FP + 4]
        fbs = scratch[2 * FP + 4:2 * FP + 6]
        sems = scratch[2 * FP + 6:]
        cid = lax.axis_index("c")
        sid = lax.axis_index("s")
        wid = cid * 16 + sid
        fg = wid % n_fg
        es = wid // n_fg
        base = es * edges

        def issue(i, slot):
            b = base + i * ch
            pltpu.async_copy(src_hbm.at[pl.ds(b, ch)], ibs[slot], sems[slot])
            pltpu.async_copy(dst_hbm.at[pl.ds(b, ch)], ids[slot], sems[slot])
            pltpu.async_copy(norm_hbm.at[pl.ds(b, ch)], fbs[slot], sems[slot])

        def drain(slot):
            pltpu.make_async_copy(src_hbm.at[pl.ds(0, ch)], ibs[slot],
                                  sems[slot]).wait()
            pltpu.make_async_copy(dst_hbm.at[pl.ds(0, ch)], ids[slot],
                                  sems[slot]).wait()
            pltpu.make_async_copy(norm_hbm.at[pl.ds(0, ch)], fbs[slot],
                                  sems[slot]).wait()

        def compute(slot):
            ib1, ib2, fb = ibs[slot], ids[slot], fbs[slot]

            def grp(g):
                off = g * 16
                s16 = ib1[pl.ds(off, 16)]
                d16 = ib2[pl.ds(off, 16)]
                n16 = fb[pl.ds(off, 16)]
                for j in range(FP):
                    v = plsc.load_gather(tbl[j], [s16]) * n16
                    plsc.addupdate_scatter(acc[j], [d16], v)
            plsc.parallel_loop(0, ch // 16, 1, unroll=unroll)(grp)

        issue(0, 0)
        issue(1, 1)

        for j in range(FP):
            pltpu.sync_copy(
                xwT_hbm.at[pl.ds((fg * FP + j) * _NP, _NP)], tbl[j])

        def zero_body(i):
            z = jnp.zeros((16,), jnp.float32)
            for j in range(FP):
                acc[j][pl.ds(i * 16, 16)] = z
        plsc.parallel_loop(0, _NP // 16, 1, unroll=8)(zero_body)

        def pair_body(kk, _):
            i = kk * 2
            for slot in (0, 1):
                drain(slot)
                compute(slot)

                @pl.when(i + 2 + slot < nch)
                def _(slot=slot):
                    issue(i + 2 + slot, slot)
            return 0
        lax.fori_loop(0, nch // 2, pair_body, 0)

        for j in range(FP):
            pltpu.sync_copy(
                acc[j],
                out_hbm.at[pl.ds((es * F + fg * FP + j) * _NP, _NP)])

    return k


def _mid_body(p_ref, xw_ref, sw_ref, b1_ref, g1_ref, be1_ref, w2_ref, o_ref):
    h = jnp.sum(p_ref[...], axis=0) + sw_ref[...] * xw_ref[...] + b1_ref[...]
    h = jnp.maximum(h, 0.0)
    mu = jnp.mean(h, axis=0, keepdims=True)
    var = jnp.mean((h - mu) ** 2, axis=0, keepdims=True)
    hn = (h - mu) * lax.rsqrt(var + 1e-5) * g1_ref[...] + be1_ref[...]
    o_ref[...] = lax.dot_general(w2_ref[...], hn, (((1,), (0,)), ((), ())),
                                 preferred_element_type=jnp.float32)


def _mid(parts, xw1T, selfw, b1, g1, be1, W2):
    n_es = parts.shape[0]
    return pl.pallas_call(
        _mid_body,
        grid=(_NBLK,),
        in_specs=[
            pl.BlockSpec((n_es, _H, _R), lambda i: (0, 0, i)),
            pl.BlockSpec((_H, _R), lambda i: (0, i)),
            pl.BlockSpec((1, _R), lambda i: (0, i)),
            pl.BlockSpec((_H, 1), lambda i: (0, 0)),
            pl.BlockSpec((_H, 1), lambda i: (0, 0)),
            pl.BlockSpec((_H, 1), lambda i: (0, 0)),
            pl.BlockSpec((32, _H), lambda i: (0, 0)),
        ],
        out_specs=pl.BlockSpec((32, _R), lambda i: (0, i)),
        out_shape=jax.ShapeDtypeStruct((32, _NP), jnp.float32),
    )(parts, xw1T, selfw.reshape(1, _NP), b1.reshape(_H, 1),
      g1.reshape(_H, 1), be1.reshape(_H, 1), W2)


def _final_body(p_ref, xw_ref, sw_ref, b2_ref, g2_ref, be2_ref, bt_ref,
                wfc_ref, bfc_ref, x3_ref, x4_ref, acc_ref):
    i = pl.program_id(0)
    h = jnp.sum(p_ref[...], axis=0) + sw_ref[...] * xw_ref[...] + b2_ref[...]
    h = jnp.maximum(h, 0.0)
    mu = jnp.mean(h, axis=0, keepdims=True)
    var = jnp.mean((h - mu) ** 2, axis=0, keepdims=True)
    x3 = (h - mu) * lax.rsqrt(var + 1e-5) * g2_ref[...] + be2_ref[...]
    x3_ref[...] = x3

    bb = bt_ref[0]
    gid = lax.broadcasted_iota(jnp.int32, (_G, _R), 0)
    onehot = jnp.where(bb == gid, 1.0, 0.0)
    aug = jnp.concatenate([x3, jnp.ones((1, _R), jnp.float32)], axis=0)
    part = lax.dot_general(onehot, aug, (((1,), (1,)), ((), ())),
                           preferred_element_type=jnp.float32)

    @pl.when(i == 0)
    def _():
        acc_ref[...] = jnp.zeros_like(acc_ref)
    acc_ref[...] = acc_ref[...] + part

    @pl.when(i == _NBLK - 1)
    def _():
        a = acc_ref[...]
        cnt = jnp.maximum(a[:, 32:33], 1.0)
        mean = a[:, :32] / cnt
        x4_ref[...] = lax.dot_general(
            mean, wfc_ref[...], (((1,), (1,)), ((), ())),
            preferred_element_type=jnp.float32) + bfc_ref[...]


def _final(parts, xw2T, selfw, b2, g2, be2, batch_p, Wfc, bfc):
    n_es = parts.shape[0]
    return pl.pallas_call(
        _final_body,
        grid=(_NBLK,),
        in_specs=[
            pl.BlockSpec((n_es, 32, _R), lambda i: (0, 0, i)),
            pl.BlockSpec((32, _R), lambda i: (0, i)),
            pl.BlockSpec((1, _R), lambda i: (0, i)),
            pl.BlockSpec((32, 1), lambda i: (0, 0)),
            pl.BlockSpec((32, 1), lambda i: (0, 0)),
            pl.BlockSpec((32, 1), lambda i: (0, 0)),
            pl.BlockSpec((1, 1, _R), lambda i: (i, 0, 0)),
            pl.BlockSpec((_OUT, 32), lambda i: (0, 0)),
            pl.BlockSpec((1, _OUT), lambda i: (0, 0)),
        ],
        out_specs=[
            pl.BlockSpec((32, _R), lambda i: (0, i)),
            pl.BlockSpec((_G, _OUT), lambda i: (0, 0)),
        ],
        out_shape=[
            jax.ShapeDtypeStruct((32, _NP), jnp.float32),
            jax.ShapeDtypeStruct((_G, _OUT), jnp.float32),
        ],
        scratch_shapes=[pltpu.VMEM((_G, 33), jnp.float32)],
    )(parts, xw2T, selfw.reshape(1, _NP), b2.reshape(32, 1),
      g2.reshape(32, 1), be2.reshape(32, 1),
      batch_p.reshape(_NBLK, 1, _R), Wfc, bfc.reshape(1, _OUT))


_make_agg = functools.lru_cache(maxsize=None)(_make_agg)


def kernel(x, edge_index, edge_attr, batch, W_att, b_att, W1, b1, g1, be1,
           W2, b2, g2, be2, Wfc, bfc):
    src, dst = edge_index[0], edge_index[1]
    x_p = jnp.zeros((_NP, _D), jnp.float32).at[:_N].set(x)
    batch_p = jnp.full((_NP,), _G, jnp.int32).at[:_N].set(batch)

    xw1T = _attention(x_p, W_att, b_att, W1)
    selfw, norm = _norm_sc(src, dst, edge_attr)
    p1 = _make_agg(_H, n_fg=4, n_es=8, ch=4000)(
        xw1T.reshape(-1), src, dst, norm).reshape(8, _H, _NP)
    xw2T = _mid(p1, xw1T, selfw, b1, g1, be1, W2)
    p2 = _make_agg(32, n_fg=8, n_es=4, ch=4000)(
        xw2T.reshape(-1), src, dst, norm).reshape(4, 32, _NP)
    x3T, x4 = _final(p2, xw2T, selfw, b2, g2, be2, batch_p, Wfc, bfc)
    x3 = x3T[:, :_N].T
    return (x3, x4)


# speedup vs baseline: 1.0619x; 1.0619x over previous
import functools

import jax
import jax.numpy as jnp
from jax import lax
from jax.experimental import pallas as pl
from jax.experimental.pallas import tpu as pltpu
from jax.experimental.pallas import tpu_sc as plsc

_N = 10000
_E = 320000
_D = 128
_H = 16
_G = 128
_OUT = 10
_NP = 10240
_R = 1024
_NBLK = _NP // _R

_SC_PARAMS = pltpu.CompilerParams(needs_layout_passes=False)


def _sc_mesh():
    return plsc.VectorSubcoreMesh(core_axis_name="c", subcore_axis_name="s",
                                  num_cores=2, num_subcores=16)


def _rsqrt_newton(x):
    i = lax.bitcast_convert_type(x, jnp.int32)
    i = jnp.int32(0x5F3759DF) - lax.shift_right_arithmetic(i, 1)
    y = lax.bitcast_convert_type(i, jnp.float32)
    for _ in range(3):
        y = y * (1.5 - 0.5 * x * y * y)
    return y


def _att_body(x_ref, wa_ref, ba_ref, w1_ref, o_ref):
    lt = lax.dot_general(wa_ref[...], x_ref[...], (((1,), (1,)), ((), ())),
                         preferred_element_type=jnp.float32)
    lt = lt + ba_ref[...]
    m = jnp.max(lt, axis=0, keepdims=True)
    e = jnp.exp(lt - m)
    aT = e / jnp.sum(e, axis=0, keepdims=True)
    o_ref[...] = lax.dot_general(w1_ref[...], aT, (((1,), (0,)), ((), ())),
                                 preferred_element_type=jnp.float32)


def _attention(x_p, W_att, b_att, W1):
    return pl.pallas_call(
        _att_body,
        grid=(_NBLK,),
        in_specs=[
            pl.BlockSpec((_R, _D), lambda i: (i, 0)),
            pl.BlockSpec((_H, _D), lambda i: (0, 0)),
            pl.BlockSpec((_H, 1), lambda i: (0, 0)),
            pl.BlockSpec((_H, _H), lambda i: (0, 0)),
        ],
        out_specs=pl.BlockSpec((_H, _R), lambda i: (0, i)),
        out_shape=jax.ShapeDtypeStruct((_H, _NP), jnp.float32),
    )(x_p, W_att, b_att.reshape(_H, 1), W1)


def _norm_sc(src, dst, ew):
    deg_ch, deg_nch = 4000, 5
    nm_ch, nm_nch = 2000, 5

    @functools.partial(
        pl.kernel,
        out_type=[
            jax.ShapeDtypeStruct((_NP,), jnp.float32),
            jax.ShapeDtypeStruct((_E,), jnp.float32),
        ],
        mesh=_sc_mesh(),
        scratch_types=[
            pltpu.VMEM((_NP,), jnp.float32),
            pltpu.VMEM((_NP,), jnp.float32),
            pltpu.VMEM((deg_ch,), jnp.int32),
            pltpu.VMEM((nm_ch,), jnp.int32),
            pltpu.VMEM((deg_ch,), jnp.float32),
            pltpu.VMEM((nm_ch,), jnp.float32),
            pltpu.VMEM((640,), jnp.float32),
            pltpu.VMEM((640,), jnp.float32),
            pltpu.VMEM_SHARED((16 * _NP,), jnp.float32),
            pltpu.VMEM_SHARED((_NP,), jnp.float32),
        ],
        compiler_params=_SC_PARAMS,
    )
    def k(src_hbm, dst_hbm, ew_hbm, selfw_hbm, norm_hbm,
          deg_v, dinv_v, ib1, ib2, fb1, nb, a640, t640, sm_degs, sm_dinv):
        cid = lax.axis_index("c")
        sid = lax.axis_index("s")
        wid = cid * 16 + sid

        def zero_body(i):
            deg_v[pl.ds(i * 16, 16)] = jnp.zeros((16,), jnp.float32)
        plsc.parallel_loop(0, _NP // 16, 1, unroll=8)(zero_body)

        def deg_chunk(i, _):
            b = sid * 20000 + i * deg_ch
            pltpu.sync_copy(dst_hbm.at[pl.ds(b, deg_ch)], ib1)
            pltpu.sync_copy(ew_hbm.at[pl.ds(b, deg_ch)], fb1)

            def grp(j):
                d16 = ib1[pl.ds(j * 16, 16)]
                w16 = fb1[pl.ds(j * 16, 16)]
                plsc.addupdate_scatter(deg_v, [d16], w16)
            plsc.parallel_loop(0, deg_ch // 16, 1, unroll=10)(grp)
            return 0
        lax.fori_loop(0, deg_nch, deg_chunk, 0)

        pltpu.sync_copy(deg_v, sm_degs.at[pl.ds(sid * _NP, _NP)])
        plsc.subcore_barrier()

        pltpu.sync_copy(sm_degs.at[pl.ds(sid * 640, 640)], a640)
        for t in range(1, 16):
            pltpu.sync_copy(sm_degs.at[pl.ds(t * _NP + sid * 640, 640)], t640)

            def add_body(kk, _t=t):
                a640[pl.ds(kk * 16, 16)] = (a640[pl.ds(kk * 16, 16)] +
                                            t640[pl.ds(kk * 16, 16)])
            plsc.parallel_loop(0, 40, 1, unroll=8)(add_body)

        def dinv_body(j):
            d = a640[pl.ds(j * 16, 16)] + 1.0
            y = _rsqrt_newton(d)
            nb[pl.ds(j * 16, 16)] = y
            fb1[pl.ds(1024 + j * 16, 16)] = y * y
        plsc.parallel_loop(0, 40, 1, unroll=8)(dinv_body)
        pltpu.sync_copy(nb.at[pl.ds(0, 640)],
                        sm_dinv.at[pl.ds(sid * 640, 640)])

        @pl.when(cid == 0)
        def _():
            pltpu.sync_copy(fb1.at[pl.ds(1024, 640)],
                            selfw_hbm.at[pl.ds(sid * 640, 640)])
        plsc.subcore_barrier()
        pltpu.sync_copy(sm_dinv, dinv_v)

        def nm_chunk(i, _):
            b = wid * 10000 + i * nm_ch
            pltpu.sync_copy(src_hbm.at[pl.ds(b, nm_ch)], ib2)
            pltpu.sync_copy(dst_hbm.at[pl.ds(b, nm_ch)],
                            ib1.at[pl.ds(0, nm_ch)])
            pltpu.sync_copy(ew_hbm.at[pl.ds(b, nm_ch)],
                            fb1.at[pl.ds(0, nm_ch)])

            def grp(j):
                s16 = ib2[pl.ds(j * 16, 16)]
                d16 = ib1[pl.ds(j * 16, 16)]
                w16 = fb1[pl.ds(j * 16, 16)]
                nv = (plsc.load_gather(dinv_v, [s16]) * w16 *
                      plsc.load_gather(dinv_v, [d16]))
                nb[pl.ds(j * 16, 16)] = nv
            plsc.parallel_loop(0, nm_ch // 16, 1, unroll=10)(grp)
            pltpu.sync_copy(nb, norm_hbm.at[pl.ds(b, nm_ch)])
            return 0
        lax.fori_loop(0, nm_nch, nm_chunk, 0)

    return k(src, dst, ew)


def _make_agg(F, n_fg, n_es, ch, unroll=10):
    FP = F // n_fg
    edges = _E // n_es
    nch = edges // ch
    assert ch % (16 * unroll) == 0 and edges % ch == 0

    @functools.partial(
        pl.kernel,
        out_type=jax.ShapeDtypeStruct((n_es * F * _NP,), jnp.float32),
        mesh=_sc_mesh(),
        scratch_types=(
            [pltpu.VMEM((_NP,), jnp.float32) for _ in range(FP)] +
            [pltpu.VMEM((_NP,), jnp.float32) for _ in range(FP)] +
            [pltpu.VMEM((ch,), jnp.int32) for _ in range(2)] +
            [pltpu.VMEM((ch,), jnp.int32) for _ in range(2)] +
            [pltpu.VMEM((ch,), jnp.float32) for _ in range(2)] +
            [pltpu.SemaphoreType.DMA, pltpu.SemaphoreType.DMA]
        ),
        compiler_params=_SC_PARAMS,
    )
    def k(xwT_hbm, src_hbm, dst_hbm, norm_hbm, out_hbm, *scratch):
        tbl = scratch[:FP]
        acc = scratch[FP:2 * FP]
        ibs = scratch[2 * FP:2 * FP + 2]
        ids = scratch[2 * FP + 2:2 * FP + 4]
        fbs = scratch[2 * FP + 4:2 * FP + 6]
        sems = scratch[2 * FP + 6:]
        cid = lax.axis_index("c")
        sid = lax.axis_index("s")
        wid = cid * 16 + sid
        fg = wid % n_fg
        es = wid // n_fg
        base = es * edges

        def issue(i, slot):
            b = base + i * ch
            pltpu.async_copy(src_hbm.at[pl.ds(b, ch)], ibs[slot], sems[slot])
            pltpu.async_copy(dst_hbm.at[pl.ds(b, ch)], ids[slot], sems[slot])
            pltpu.async_copy(norm_hbm.at[pl.ds(b, ch)], fbs[slot], sems[slot])

        def drain(slot):
            pltpu.make_async_copy(src_hbm.at[pl.ds(0, ch)], ibs[slot],
                                  sems[slot]).wait()
            pltpu.make_async_copy(dst_hbm.at[pl.ds(0, ch)], ids[slot],
                                  sems[slot]).wait()
            pltpu.make_async_copy(norm_hbm.at[pl.ds(0, ch)], fbs[slot],
                                  sems[slot]).wait()

        def compute(slot):
            ib1, ib2, fb = ibs[slot], ids[slot], fbs[slot]

            def grp(g):
                off = g * 16
                s16 = ib1[pl.ds(off, 16)]
                d16 = ib2[pl.ds(off, 16)]
                n16 = fb[pl.ds(off, 16)]
                for j in range(FP):
                    v = plsc.load_gather(tbl[j], [s16]) * n16
                    plsc.addupdate_scatter(acc[j], [d16], v)
            plsc.parallel_loop(0, ch // 16, 1, unroll=unroll)(grp)

        issue(0, 0)
        issue(1, 1)

        for j in range(FP):
            pltpu.sync_copy(
                xwT_hbm.at[pl.ds((fg * FP + j) * _NP, _NP)], tbl[j])

        def zero_body(i):
            z = jnp.zeros((16,), jnp.float32)
            for j in range(FP):
                acc[j][pl.ds(i * 16, 16)] = z
        plsc.parallel_loop(0, _NP // 16, 1, unroll=8)(zero_body)

        def pair_body(kk, _):
            i = kk * 2
            for slot in (0, 1):
                drain(slot)
                compute(slot)

                @pl.when(i + 2 + slot < nch)
                def _(slot=slot):
                    issue(i + 2 + slot, slot)
            return 0
        lax.fori_loop(0, nch // 2, pair_body, 0)

        for j in range(FP):
            pltpu.sync_copy(
                acc[j],
                out_hbm.at[pl.ds((es * F + fg * FP + j) * _NP, _NP)])

    return k


def _mid_body(p_ref, xw_ref, sw_ref, b1_ref, g1_ref, be1_ref, w2_ref, o_ref):
    h = jnp.sum(p_ref[...], axis=0) + sw_ref[...] * xw_ref[...] + b1_ref[...]
    h = jnp.maximum(h, 0.0)
    mu = jnp.mean(h, axis=0, keepdims=True)
    var = jnp.mean((h - mu) ** 2, axis=0, keepdims=True)
    hn = (h - mu) * lax.rsqrt(var + 1e-5) * g1_ref[...] + be1_ref[...]
    o_ref[...] = lax.dot_general(w2_ref[...], hn, (((1,), (0,)), ((), ())),
                                 preferred_element_type=jnp.float32)


def _mid(parts, xw1T, selfw, b1, g1, be1, W2):
    n_es = parts.shape[0]
    return pl.pallas_call(
        _mid_body,
        grid=(_NBLK,),
        in_specs=[
            pl.BlockSpec((n_es, _H, _R), lambda i: (0, 0, i)),
            pl.BlockSpec((_H, _R), lambda i: (0, i)),
            pl.BlockSpec((1, _R), lambda i: (0, i)),
            pl.BlockSpec((_H, 1), lambda i: (0, 0)),
            pl.BlockSpec((_H, 1), lambda i: (0, 0)),
            pl.BlockSpec((_H, 1), lambda i: (0, 0)),
            pl.BlockSpec((32, _H), lambda i: (0, 0)),
        ],
        out_specs=pl.BlockSpec((32, _R), lambda i: (0, i)),
        out_shape=jax.ShapeDtypeStruct((32, _NP), jnp.float32),
    )(parts, xw1T, selfw.reshape(1, _NP), b1.reshape(_H, 1),
      g1.reshape(_H, 1), be1.reshape(_H, 1), W2)


def _final_body(p_ref, xw_ref, sw_ref, b2_ref, g2_ref, be2_ref, bt_ref,
                wfc_ref, bfc_ref, x3_ref, x4_ref, acc_ref):
    i = pl.program_id(0)
    h = jnp.sum(p_ref[...], axis=0) + sw_ref[...] * xw_ref[...] + b2_ref[...]
    h = jnp.maximum(h, 0.0)
    mu = jnp.mean(h, axis=0, keepdims=True)
    var = jnp.mean((h - mu) ** 2, axis=0, keepdims=True)
    x3 = (h - mu) * lax.rsqrt(var + 1e-5) * g2_ref[...] + be2_ref[...]
    x3_ref[...] = x3

    bb = bt_ref[0]
    gid = lax.broadcasted_iota(jnp.int32, (_G, _R), 0)
    onehot = jnp.where(bb == gid, 1.0, 0.0)
    aug = jnp.concatenate([x3, jnp.ones((1, _R), jnp.float32)], axis=0)
    part = lax.dot_general(onehot, aug, (((1,), (1,)), ((), ())),
                           preferred_element_type=jnp.float32)

    @pl.when(i == 0)
    def _():
        acc_ref[...] = jnp.zeros_like(acc_ref)
    acc_ref[...] = acc_ref[...] + part

    @pl.when(i == _NBLK - 1)
    def _():
        a = acc_ref[...]
        cnt = jnp.maximum(a[:, 32:33], 1.0)
        mean = a[:, :32] / cnt
        x4_ref[...] = lax.dot_general(
            mean, wfc_ref[...], (((1,), (1,)), ((), ())),
            preferred_element_type=jnp.float32) + bfc_ref[...]


def _final(parts, xw2T, selfw, b2, g2, be2, batch_p, Wfc, bfc):
    n_es = parts.shape[0]
    return pl.pallas_call(
        _final_body,
        grid=(_NBLK,),
        in_specs=[
            pl.BlockSpec((n_es, 32, _R), lambda i: (0, 0, i)),
            pl.BlockSpec((32, _R), lambda i: (0, i)),
            pl.BlockSpec((1, _R), lambda i: (0, i)),
            pl.BlockSpec((32, 1), lambda i: (0, 0)),
            pl.BlockSpec((32, 1), lambda i: (0, 0)),
            pl.BlockSpec((32, 1), lambda i: (0, 0)),
            pl.BlockSpec((1, 1, _R), lambda i: (i, 0, 0)),
            pl.BlockSpec((_OUT, 32), lambda i: (0, 0)),
            pl.BlockSpec((1, _OUT), lambda i: (0, 0)),
        ],
        out_specs=[
            pl.BlockSpec((32, _R), lambda i: (0, i)),
            pl.BlockSpec((_G, _OUT), lambda i: (0, 0)),
        ],
        out_shape=[
            jax.ShapeDtypeStruct((32, _NP), jnp.float32),
            jax.ShapeDtypeStruct((_G, _OUT), jnp.float32),
        ],
        scratch_shapes=[pltpu.VMEM((_G, 33), jnp.float32)],
    )(parts, xw2T, selfw.reshape(1, _NP), b2.reshape(32, 1),
      g2.reshape(32, 1), be2.reshape(32, 1),
      batch_p.reshape(_NBLK, 1, _R), Wfc, bfc.reshape(1, _OUT))


_make_agg = functools.lru_cache(maxsize=None)(_make_agg)


def kernel(x, edge_index, edge_attr, batch, W_att, b_att, W1, b1, g1, be1,
           W2, b2, g2, be2, Wfc, bfc):
    src, dst = edge_index[0], edge_index[1]
    x_p = jnp.zeros((_NP, _D), jnp.float32).at[:_N].set(x)
    batch_p = jnp.full((_NP,), _G, jnp.int32).at[:_N].set(batch)

    xw1T = _attention(x_p, W_att, b_att, W1)
    selfw, norm = _norm_sc(src, dst, edge_attr)
    p1 = _make_agg(_H, n_fg=4, n_es=8, ch=4000)(
        xw1T.reshape(-1), src, dst, norm).reshape(8, _H, _NP)
    xw2T = _mid(p1, xw1T, selfw, b1, g1, be1, W2)
    p2 = _make_agg(32, n_fg=8, n_es=4, ch=4000)(
        xw2T.reshape(-1), src, dst, norm).reshape(4, 32, _NP)
    x3T, x4 = _final(p2, xw2T, selfw, b2, g2, be2, batch_p, Wfc, bfc)
    x3 = x3T[:, :_N].T
    return (x3, x4)


# speedup vs baseline: 1.0804x; 1.0174x over previous
import functools

import jax
import jax.numpy as jnp
from jax import lax
from jax.experimental import pallas as pl
from jax.experimental.pallas import tpu as pltpu
from jax.experimental.pallas import tpu_sc as plsc

_N = 10000
_E = 320000
_D = 128
_H = 16
_G = 128
_OUT = 10
_NP = 10240
_R = 1024
_NBLK = _NP // _R

_SC_PARAMS = pltpu.CompilerParams(needs_layout_passes=False)


def _sc_mesh():
    return plsc.VectorSubcoreMesh(core_axis_name="c", subcore_axis_name="s",
                                  num_cores=2, num_subcores=16)


def _rsqrt_newton(x):
    i = lax.bitcast_convert_type(x, jnp.int32)
    i = jnp.int32(0x5F3759DF) - lax.shift_right_arithmetic(i, 1)
    y = lax.bitcast_convert_type(i, jnp.float32)
    for _ in range(3):
        y = y * (1.5 - 0.5 * x * y * y)
    return y


def _att_body(x_ref, wa_ref, ba_ref, w1_ref, o_ref):
    lt = lax.dot_general(wa_ref[...], x_ref[...], (((1,), (1,)), ((), ())),
                         preferred_element_type=jnp.float32)
    lt = lt + ba_ref[...]
    m = jnp.max(lt, axis=0, keepdims=True)
    e = jnp.exp(lt - m)
    aT = e / jnp.sum(e, axis=0, keepdims=True)
    o_ref[...] = lax.dot_general(w1_ref[...], aT, (((1,), (0,)), ((), ())),
                                 preferred_element_type=jnp.float32)


def _attention(x_p, W_att, b_att, W1):
    return pl.pallas_call(
        _att_body,
        grid=(_NBLK,),
        in_specs=[
            pl.BlockSpec((_R, _D), lambda i: (i, 0)),
            pl.BlockSpec((_H, _D), lambda i: (0, 0)),
            pl.BlockSpec((_H, 1), lambda i: (0, 0)),
            pl.BlockSpec((_H, _H), lambda i: (0, 0)),
        ],
        out_specs=pl.BlockSpec((_H, _R), lambda i: (0, i)),
        out_shape=jax.ShapeDtypeStruct((_H, _NP), jnp.float32),
    )(x_p, W_att, b_att.reshape(_H, 1), W1)


def _norm_agg16(xw1T_flat, src, dst, ew):
    deg_ch, deg_nch = 4000, 5
    nm_ch, nm_nch = 2000, 5
    F, n_fg, n_es, ch, unroll = _H, 4, 8, 4000, 10
    FP = F // n_fg
    edges = _E // n_es
    nch = edges // ch
    half = _E // 2

    @functools.partial(
        pl.kernel,
        out_type=[
            jax.ShapeDtypeStruct((_NP,), jnp.float32),
            jax.ShapeDtypeStruct((_E,), jnp.float32),
            jax.ShapeDtypeStruct((n_es * F * _NP,), jnp.float32),
        ],
        mesh=_sc_mesh(),
        scratch_types=(
            [pltpu.VMEM((_NP,), jnp.float32) for _ in range(FP)] +
            [pltpu.VMEM((_NP,), jnp.float32) for _ in range(FP)] +
            [pltpu.VMEM((_NP,), jnp.float32),
             pltpu.VMEM((ch,), jnp.int32), pltpu.VMEM((ch,), jnp.int32),
             pltpu.VMEM((ch,), jnp.int32), pltpu.VMEM((ch,), jnp.int32),
             pltpu.VMEM((ch,), jnp.float32), pltpu.VMEM((ch,), jnp.float32),
             pltpu.VMEM((nm_ch,), jnp.float32),
             pltpu.VMEM((640,), jnp.float32), pltpu.VMEM((640,), jnp.float32),
             pltpu.SemaphoreType.DMA, pltpu.SemaphoreType.DMA,
             pltpu.VMEM_SHARED((16 * _NP,), jnp.float32),
             pltpu.VMEM_SHARED((_NP,), jnp.float32)]
        ),
        compiler_params=_SC_PARAMS,
    )
    def k(xwT_hbm, src_hbm, dst_hbm, ew_hbm, selfw_hbm, norm_hbm, out_hbm,
          *sc):
        tbl, acc = sc[0:FP], sc[FP:2 * FP]
        dinv_v = sc[2 * FP]
        ibs = sc[2 * FP + 1:2 * FP + 3]
        ids = sc[2 * FP + 3:2 * FP + 5]
        fbs = sc[2 * FP + 5:2 * FP + 7]
        nb, a640, t640 = sc[2 * FP + 7:2 * FP + 10]
        sems = sc[2 * FP + 10:2 * FP + 12]
        sm_degs, sm_dinv = sc[2 * FP + 12:]
        cid = lax.axis_index("c")
        sid = lax.axis_index("s")
        wid = cid * 16 + sid
        fg = wid % n_fg
        es = wid // n_fg
        es_local = sid // n_fg

        def zero_body(i):
            z = jnp.zeros((16,), jnp.float32)
            for j in range(FP):
                acc[j][pl.ds(i * 16, 16)] = z
        plsc.parallel_loop(0, _NP // 16, 1, unroll=8)(zero_body)

        def dissue(i, slot):
            b = sid * 20000 + i * deg_ch
            pltpu.async_copy(dst_hbm.at[pl.ds(b, deg_ch)], ibs[slot],
                             sems[slot])
            pltpu.async_copy(ew_hbm.at[pl.ds(b, deg_ch)], fbs[slot],
                             sems[slot])

        def ddrain(slot):
            pltpu.make_async_copy(dst_hbm.at[pl.ds(0, deg_ch)], ibs[slot],
                                  sems[slot]).wait()
            pltpu.make_async_copy(ew_hbm.at[pl.ds(0, deg_ch)], fbs[slot],
                                  sems[slot]).wait()

        dissue(0, 0)
        for i in range(deg_nch):
            slot = i % 2
            if i + 1 < deg_nch:
                dissue(i + 1, (i + 1) % 2)
            ddrain(slot)

            def dgrp(j, slot=slot):
                d16 = ibs[slot][pl.ds(j * 16, 16)]
                w16 = fbs[slot][pl.ds(j * 16, 16)]
                plsc.addupdate_scatter(acc[0], [d16], w16)
            plsc.parallel_loop(0, deg_ch // 16, 1, unroll=10)(dgrp)

        pltpu.sync_copy(acc[0], sm_degs.at[pl.ds(sid * _NP, _NP)])
        plsc.subcore_barrier()

        def zero0(i):
            acc[0][pl.ds(i * 16, 16)] = jnp.zeros((16,), jnp.float32)
        plsc.parallel_loop(0, _NP // 16, 1, unroll=8)(zero0)

        pltpu.sync_copy(sm_degs.at[pl.ds(sid * 640, 640)], a640)
        for t in range(1, 16):
            pltpu.sync_copy(sm_degs.at[pl.ds(t * _NP + sid * 640, 640)], t640)

            def add_body(kk, _t=t):
                a640[pl.ds(kk * 16, 16)] = (a640[pl.ds(kk * 16, 16)] +
                                            t640[pl.ds(kk * 16, 16)])
            plsc.parallel_loop(0, 40, 1, unroll=8)(add_body)

        def dinv_body(j):
            d = a640[pl.ds(j * 16, 16)] + 1.0
            y = _rsqrt_newton(d)
            nb[pl.ds(j * 16, 16)] = y
            fbs[0][pl.ds(1024 + j * 16, 16)] = y * y
        plsc.parallel_loop(0, 40, 1, unroll=8)(dinv_body)
        pltpu.sync_copy(nb.at[pl.ds(0, 640)],
                        sm_dinv.at[pl.ds(sid * 640, 640)])

        @pl.when(cid == 0)
        def _():
            pltpu.sync_copy(fbs[0].at[pl.ds(1024, 640)],
                            selfw_hbm.at[pl.ds(sid * 640, 640)])
        plsc.subcore_barrier()
        pltpu.sync_copy(sm_dinv, dinv_v)

        def nissue(i, slot):
            b = wid * 10000 + i * nm_ch
            pltpu.async_copy(src_hbm.at[pl.ds(b, nm_ch)],
                             ibs[slot].at[pl.ds(0, nm_ch)], sems[slot])
            pltpu.async_copy(dst_hbm.at[pl.ds(b, nm_ch)],
                             ids[slot].at[pl.ds(0, nm_ch)], sems[slot])
            pltpu.async_copy(ew_hbm.at[pl.ds(b, nm_ch)],
                             fbs[slot].at[pl.ds(0, nm_ch)], sems[slot])

        def ndrain(slot):
            for hbm, buf in ((src_hbm, ibs[slot]), (dst_hbm, ids[slot]),
                             (ew_hbm, fbs[slot])):
                pltpu.make_async_copy(hbm.at[pl.ds(0, nm_ch)],
                                      buf.at[pl.ds(0, nm_ch)],
                                      sems[slot]).wait()

        nissue(0, 0)
        for i in range(nm_nch):
            slot = i % 2
            if i + 1 < nm_nch:
                nissue(i + 1, (i + 1) % 2)
            ndrain(slot)

            def ngrp(j, slot=slot):
                s16 = ibs[slot][pl.ds(j * 16, 16)]
                d16 = ids[slot][pl.ds(j * 16, 16)]
                w16 = fbs[slot][pl.ds(j * 16, 16)]
                nv = (plsc.load_gather(dinv_v, [s16]) * w16 *
                      plsc.load_gather(dinv_v, [d16]))
                nb[pl.ds(j * 16, 16)] = nv
            plsc.parallel_loop(0, nm_ch // 16, 1, unroll=10)(ngrp)
            pltpu.sync_copy(nb, norm_hbm.at[pl.ds(wid * 10000 + i * nm_ch,
                                                  nm_ch)])
        plsc.subcore_barrier()

        for j in range(FP):
            pltpu.sync_copy(
                xwT_hbm.at[pl.ds((fg * FP + j) * _NP, _NP)], tbl[j])

        def aissue(i, slot):
            b = es * edges + i * ch
            pltpu.async_copy(src_hbm.at[pl.ds(b, ch)], ibs[slot], sems[slot])
            pltpu.async_copy(dst_hbm.at[pl.ds(b, ch)], ids[slot], sems[slot])
            pltpu.async_copy(norm_hbm.at[pl.ds(b, ch)], fbs[slot], sems[slot])

        def adrain(slot):
            pltpu.make_async_copy(src_hbm.at[pl.ds(0, ch)], ibs[slot],
                                  sems[slot]).wait()
            pltpu.make_async_copy(dst_hbm.at[pl.ds(0, ch)], ids[slot],
                                  sems[slot]).wait()
            pltpu.make_async_copy(norm_hbm.at[pl.ds(0, ch)], fbs[slot],
                                  sems[slot]).wait()

        aissue(0, 0)
        aissue(1, 1)

        def pair_body(kk, _):
            i = kk * 2
            for slot in (0, 1):
                adrain(slot)

                def grp(g, slot=slot):
                    off = g * 16
                    s16 = ibs[slot][pl.ds(off, 16)]
                    d16 = ids[slot][pl.ds(off, 16)]
                    n16 = fbs[slot][pl.ds(off, 16)]
                    for j in range(FP):
                        v = plsc.load_gather(tbl[j], [s16]) * n16
                        plsc.addupdate_scatter(acc[j], [d16], v)
                plsc.parallel_loop(0, ch // 16, 1, unroll=unroll)(grp)

                @pl.when(i + 2 + slot < nch)
                def _(slot=slot):
                    aissue(i + 2 + slot, slot)
            return 0
        lax.fori_loop(0, nch // 2, pair_body, 0)

        for j in range(FP):
            pltpu.sync_copy(
                acc[j],
                out_hbm.at[pl.ds((es * F + fg * FP + j) * _NP, _NP)])

    return k(xw1T_flat, src, dst, ew)


def _make_agg(F, n_fg, n_es, ch, unroll=10):
    FP = F // n_fg
    edges = _E // n_es
    nch = edges // ch
    assert ch % (16 * unroll) == 0 and edges % ch == 0

    @functools.partial(
        pl.kernel,
        out_type=jax.ShapeDtypeStruct((n_es * F * _NP,), jnp.float32),
        mesh=_sc_mesh(),
        scratch_types=(
            [pltpu.VMEM((_NP,), jnp.float32) for _ in range(FP)] +
            [pltpu.VMEM((_NP,), jnp.float32) for _ in range(FP)] +
            [pltpu.VMEM((ch,), jnp.int32) for _ in range(2)] +
            [pltpu.VMEM((ch,), jnp.int32) for _ in range(2)] +
            [pltpu.VMEM((ch,), jnp.float32) for _ in range(2)] +
            [pltpu.SemaphoreType.DMA, pltpu.SemaphoreType.DMA]
        ),
        compiler_params=_SC_PARAMS,
    )
    def k(xwT_hbm, src_hbm, dst_hbm, norm_hbm, out_hbm, *scratch):
        tbl = scratch[:FP]
        acc = scratch[FP:2 * FP]
        ibs = scratch[2 * FP:2 * FP + 2]
        ids = scratch[2 * FP + 2:2 * FP + 4]
        fbs = scratch[2 * FP + 4:2 * FP + 6]
        sems = scratch[2 * FP + 6:]
        cid = lax.axis_index("c")
        sid = lax.axis_index("s")
        wid = cid * 16 + sid
        fg = wid % n_fg
        es = wid // n_fg
        base = es * edges

        def issue(i, slot):
            b = base + i * ch
            pltpu.async_copy(src_hbm.at[pl.ds(b, ch)], ibs[slot], sems[slot])
            pltpu.async_copy(dst_hbm.at[pl.ds(b, ch)], ids[slot], sems[slot])
            pltpu.async_copy(norm_hbm.at[pl.ds(b, ch)], fbs[slot], sems[slot])

        def drain(slot):
            pltpu.make_async_copy(src_hbm.at[pl.ds(0, ch)], ibs[slot],
                                  sems[slot]).wait()
            pltpu.make_async_copy(dst_hbm.at[pl.ds(0, ch)], ids[slot],
                                  sems[slot]).wait()
            pltpu.make_async_copy(norm_hbm.at[pl.ds(0, ch)], fbs[slot],
                                  sems[slot]).wait()

        def compute(slot):
            ib1, ib2, fb = ibs[slot], ids[slot], fbs[slot]

            def grp(g):
                off = g * 16
                s16 = ib1[pl.ds(off, 16)]
                d16 = ib2[pl.ds(off, 16)]
                n16 = fb[pl.ds(off, 16)]
                for j in range(FP):
                    v = plsc.load_gather(tbl[j], [s16]) * n16
                    plsc.addupdate_scatter(acc[j], [d16], v)
            plsc.parallel_loop(0, ch // 16, 1, unroll=unroll)(grp)

        issue(0, 0)
        issue(1, 1)

        for j in range(FP):
            pltpu.sync_copy(
                xwT_hbm.at[pl.ds((fg * FP + j) * _NP, _NP)], tbl[j])

        def zero_body(i):
            z = jnp.zeros((16,), jnp.float32)
            for j in range(FP):
                acc[j][pl.ds(i * 16, 16)] = z
        plsc.parallel_loop(0, _NP // 16, 1, unroll=8)(zero_body)

        def pair_body(kk, _):
            i = kk * 2
            for slot in (0, 1):
                drain(slot)
                compute(slot)

                @pl.when(i + 2 + slot < nch)
                def _(slot=slot):
                    issue(i + 2 + slot, slot)
            return 0
        lax.fori_loop(0, nch // 2, pair_body, 0)

        for j in range(FP):
            pltpu.sync_copy(
                acc[j],
                out_hbm.at[pl.ds((es * F + fg * FP + j) * _NP, _NP)])

    return k


def _mid_body(p_ref, xw_ref, sw_ref, b1_ref, g1_ref, be1_ref, w2_ref, o_ref):
    h = jnp.sum(p_ref[...], axis=0) + sw_ref[...] * xw_ref[...] + b1_ref[...]
    h = jnp.maximum(h, 0.0)
    mu = jnp.mean(h, axis=0, keepdims=True)
    var = jnp.mean((h - mu) ** 2, axis=0, keepdims=True)
    hn = (h - mu) * lax.rsqrt(var + 1e-5) * g1_ref[...] + be1_ref[...]
    o_ref[...] = lax.dot_general(w2_ref[...], hn, (((1,), (0,)), ((), ())),
                                 preferred_element_type=jnp.float32)


def _mid(parts, xw1T, selfw, b1, g1, be1, W2):
    n_es = parts.shape[0]
    return pl.pallas_call(
        _mid_body,
        grid=(_NBLK,),
        in_specs=[
            pl.BlockSpec((n_es, _H, _R), lambda i: (0, 0, i)),
            pl.BlockSpec((_H, _R), lambda i: (0, i)),
            pl.BlockSpec((1, _R), lambda i: (0, i)),
            pl.BlockSpec((_H, 1), lambda i: (0, 0)),
            pl.BlockSpec((_H, 1), lambda i: (0, 0)),
            pl.BlockSpec((_H, 1), lambda i: (0, 0)),
            pl.BlockSpec((32, _H), lambda i: (0, 0)),
        ],
        out_specs=pl.BlockSpec((32, _R), lambda i: (0, i)),
        out_shape=jax.ShapeDtypeStruct((32, _NP), jnp.float32),
    )(parts, xw1T, selfw.reshape(1, _NP), b1.reshape(_H, 1),
      g1.reshape(_H, 1), be1.reshape(_H, 1), W2)


def _final_body(p_ref, xw_ref, sw_ref, b2_ref, g2_ref, be2_ref, bt_ref,
                wfc_ref, bfc_ref, x3_ref, x4_ref, acc_ref):
    i = pl.program_id(0)
    h = jnp.sum(p_ref[...], axis=0) + sw_ref[...] * xw_ref[...] + b2_ref[...]
    h = jnp.maximum(h, 0.0)
    mu = jnp.mean(h, axis=0, keepdims=True)
    var = jnp.mean((h - mu) ** 2, axis=0, keepdims=True)
    x3 = (h - mu) * lax.rsqrt(var + 1e-5) * g2_ref[...] + be2_ref[...]
    x3_ref[...] = x3

    bb = bt_ref[0]
    gid = lax.broadcasted_iota(jnp.int32, (_G, _R), 0)
    onehot = jnp.where(bb == gid, 1.0, 0.0)
    aug = jnp.concatenate([x3, jnp.ones((1, _R), jnp.float32)], axis=0)
    part = lax.dot_general(onehot, aug, (((1,), (1,)), ((), ())),
                           preferred_element_type=jnp.float32)

    @pl.when(i == 0)
    def _():
        acc_ref[...] = jnp.zeros_like(acc_ref)
    acc_ref[...] = acc_ref[...] + part

    @pl.when(i == _NBLK - 1)
    def _():
        a = acc_ref[...]
        cnt = jnp.maximum(a[:, 32:33], 1.0)
        mean = a[:, :32] / cnt
        x4_ref[...] = lax.dot_general(
            mean, wfc_ref[...], (((1,), (1,)), ((), ())),
            preferred_element_type=jnp.float32) + bfc_ref[...]


def _final(parts, xw2T, selfw, b2, g2, be2, batch_p, Wfc, bfc):
    n_es = parts.shape[0]
    return pl.pallas_call(
        _final_body,
        grid=(_NBLK,),
        in_specs=[
            pl.BlockSpec((n_es, 32, _R), lambda i: (0, 0, i)),
            pl.BlockSpec((32, _R), lambda i: (0, i)),
            pl.BlockSpec((1, _R), lambda i: (0, i)),
            pl.BlockSpec((32, 1), lambda i: (0, 0)),
            pl.BlockSpec((32, 1), lambda i: (0, 0)),
            pl.BlockSpec((32, 1), lambda i: (0, 0)),
            pl.BlockSpec((1, 1, _R), lambda i: (i, 0, 0)),
            pl.BlockSpec((_OUT, 32), lambda i: (0, 0)),
            pl.BlockSpec((1, _OUT), lambda i: (0, 0)),
        ],
        out_specs=[
            pl.BlockSpec((32, _R), lambda i: (0, i)),
            pl.BlockSpec((_G, _OUT), lambda i: (0, 0)),
        ],
        out_shape=[
            jax.ShapeDtypeStruct((32, _NP), jnp.float32),
            jax.ShapeDtypeStruct((_G, _OUT), jnp.float32),
        ],
        scratch_shapes=[pltpu.VMEM((_G, 33), jnp.float32)],
    )(parts, xw2T, selfw.reshape(1, _NP), b2.reshape(32, 1),
      g2.reshape(32, 1), be2.reshape(32, 1),
      batch_p.reshape(_NBLK, 1, _R), Wfc, bfc.reshape(1, _OUT))


_make_agg = functools.lru_cache(maxsize=None)(_make_agg)


def kernel(x, edge_index, edge_attr, batch, W_att, b_att, W1, b1, g1, be1,
           W2, b2, g2, be2, Wfc, bfc):
    src, dst = edge_index[0], edge_index[1]
    x_p = jnp.zeros((_NP, _D), jnp.float32).at[:_N].set(x)
    batch_p = jnp.full((_NP,), _G, jnp.int32).at[:_N].set(batch)

    xw1T = _attention(x_p, W_att, b_att, W1)
    selfw, norm, p1f = _norm_agg16(xw1T.reshape(-1), src, dst, edge_attr)
    p1 = p1f.reshape(8, _H, _NP)
    xw2T = _mid(p1, xw1T, selfw, b1, g1, be1, W2)
    p2 = _make_agg(32, n_fg=8, n_es=4, ch=4000)(
        xw2T.reshape(-1), src, dst, norm).reshape(4, 32, _NP)
    x3T, x4 = _final(p2, xw2T, selfw, b2, g2, be2, batch_p, Wfc, bfc)
    x3 = x3T[:, :_N].T
    return (x3, x4)


# speedup vs baseline: 1.1642x; 1.0776x over previous
import functools

import jax
import jax.numpy as jnp
from jax import lax
from jax.experimental import pallas as pl
from jax.experimental.pallas import tpu as pltpu
from jax.experimental.pallas import tpu_sc as plsc

_N = 10000
_E = 320000
_D = 128
_H = 16
_G = 128
_OUT = 10
_NN = 10000
_NP = 10240
_R = 1000
_NBLK = _NN // _R

_SC_PARAMS = pltpu.CompilerParams(needs_layout_passes=False)


def _sc_mesh():
    return plsc.VectorSubcoreMesh(core_axis_name="c", subcore_axis_name="s",
                                  num_cores=2, num_subcores=16)


def _rsqrt_newton(x):
    i = lax.bitcast_convert_type(x, jnp.int32)
    i = jnp.int32(0x5F3759DF) - lax.shift_right_arithmetic(i, 1)
    y = lax.bitcast_convert_type(i, jnp.float32)
    for _ in range(3):
        y = y * (1.5 - 0.5 * x * y * y)
    return y


def _att_body(x_ref, wa_ref, ba_ref, w1_ref, o_ref):
    lt = lax.dot_general(wa_ref[...], x_ref[...], (((1,), (1,)), ((), ())),
                         preferred_element_type=jnp.float32)
    lt = lt + ba_ref[...]
    m = jnp.max(lt, axis=0, keepdims=True)
    e = jnp.exp(lt - m)
    aT = e / jnp.sum(e, axis=0, keepdims=True)
    o_ref[...] = lax.dot_general(w1_ref[...], aT, (((1,), (0,)), ((), ())),
                                 preferred_element_type=jnp.float32)


def _attention(x_p, W_att, b_att, W1):
    return pl.pallas_call(
        _att_body,
        out_shape=jax.ShapeDtypeStruct((_H, _NN), jnp.float32),
    )(x_p, W_att, b_att.reshape(_H, 1), W1)


def _norm_agg16(xw1T_flat, src, dst, ew):
    deg_ch, deg_nch = 4000, 5
    nm_ch, nm_nch = 2000, 5
    F, n_fg, n_es, ch, unroll = _H, 4, 8, 4000, 10
    FP = F // n_fg
    edges = _E // n_es
    nch = edges // ch
    half = _E // 2

    @functools.partial(
        pl.kernel,
        out_type=[
            jax.ShapeDtypeStruct((_NP,), jnp.float32),
            jax.ShapeDtypeStruct((_E,), jnp.float32),
            jax.ShapeDtypeStruct((n_es * F * _NN,), jnp.float32),
        ],
        mesh=_sc_mesh(),
        scratch_types=(
            [pltpu.VMEM((_NN,), jnp.float32) for _ in range(FP)] +
            [pltpu.VMEM((_NN,), jnp.float32) for _ in range(FP)] +
            [pltpu.VMEM((_NN,), jnp.float32),
             pltpu.VMEM((ch,), jnp.int32), pltpu.VMEM((ch,), jnp.int32),
             pltpu.VMEM((ch,), jnp.int32), pltpu.VMEM((ch,), jnp.int32),
             pltpu.VMEM((ch,), jnp.float32), pltpu.VMEM((ch,), jnp.float32),
             pltpu.VMEM((nm_ch,), jnp.float32),
             pltpu.VMEM((640,), jnp.float32), pltpu.VMEM((640,), jnp.float32),
             pltpu.SemaphoreType.DMA, pltpu.SemaphoreType.DMA,
             pltpu.VMEM_SHARED((16 * _NP,), jnp.float32),
             pltpu.VMEM_SHARED((_NP,), jnp.float32)]
        ),
        compiler_params=_SC_PARAMS,
    )
    def k(xwT_hbm, src_hbm, dst_hbm, ew_hbm, selfw_hbm, norm_hbm, out_hbm,
          *sc):
        tbl, acc = sc[0:FP], sc[FP:2 * FP]
        dinv_v = sc[2 * FP]
        ibs = sc[2 * FP + 1:2 * FP + 3]
        ids = sc[2 * FP + 3:2 * FP + 5]
        fbs = sc[2 * FP + 5:2 * FP + 7]
        nb, a640, t640 = sc[2 * FP + 7:2 * FP + 10]
        sems = sc[2 * FP + 10:2 * FP + 12]
        sm_degs, sm_dinv = sc[2 * FP + 12:]
        cid = lax.axis_index("c")
        sid = lax.axis_index("s")
        wid = cid * 16 + sid
        fg = wid % n_fg
        es = wid // n_fg
        es_local = sid // n_fg

        def zero_body(i):
            z = jnp.zeros((16,), jnp.float32)
            for j in range(FP):
                acc[j][pl.ds(i * 16, 16)] = z
        plsc.parallel_loop(0, _NN // 16, 1, unroll=5)(zero_body)

        def dissue(i, slot):
            b = sid * 20000 + i * deg_ch
            pltpu.async_copy(dst_hbm.at[pl.ds(b, deg_ch)], ibs[slot],
                             sems[slot])
            pltpu.async_copy(ew_hbm.at[pl.ds(b, deg_ch)], fbs[slot],
                             sems[slot])

        def ddrain(slot):
            pltpu.make_async_copy(dst_hbm.at[pl.ds(0, deg_ch)], ibs[slot],
                                  sems[slot]).wait()
            pltpu.make_async_copy(ew_hbm.at[pl.ds(0, deg_ch)], fbs[slot],
                                  sems[slot]).wait()

        dissue(0, 0)
        for i in range(deg_nch):
            slot = i % 2
            if i + 1 < deg_nch:
                dissue(i + 1, (i + 1) % 2)
            ddrain(slot)

            def dgrp(j, slot=slot):
                d16 = ibs[slot][pl.ds(j * 16, 16)]
                w16 = fbs[slot][pl.ds(j * 16, 16)]
                plsc.addupdate_scatter(acc[0], [d16], w16)
            plsc.parallel_loop(0, deg_ch // 16, 1, unroll=10)(dgrp)

        pltpu.sync_copy(acc[0], sm_degs.at[pl.ds(sid * _NP, _NN)])
        plsc.subcore_barrier()

        def zero0(i):
            acc[0][pl.ds(i * 16, 16)] = jnp.zeros((16,), jnp.float32)
        plsc.parallel_loop(0, _NN // 16, 1, unroll=5)(zero0)

        pltpu.sync_copy(sm_degs.at[pl.ds(sid * 640, 640)], a640)
        for t in range(1, 16):
            pltpu.sync_copy(sm_degs.at[pl.ds(t * _NP + sid * 640, 640)], t640)

            def add_body(kk, _t=t):
                a640[pl.ds(kk * 16, 16)] = (a640[pl.ds(kk * 16, 16)] +
                                            t640[pl.ds(kk * 16, 16)])
            plsc.parallel_loop(0, 40, 1, unroll=8)(add_body)

        def dinv_body(j):
            d = a640[pl.ds(j * 16, 16)] + 1.0
            y = _rsqrt_newton(d)
            nb[pl.ds(j * 16, 16)] = y
            fbs[0][pl.ds(1024 + j * 16, 16)] = y * y
        plsc.parallel_loop(0, 40, 1, unroll=8)(dinv_body)
        pltpu.sync_copy(nb.at[pl.ds(0, 640)],
                        sm_dinv.at[pl.ds(sid * 640, 640)])

        @pl.when(cid == 0)
        def _():
            pltpu.sync_copy(fbs[0].at[pl.ds(1024, 640)],
                            selfw_hbm.at[pl.ds(sid * 640, 640)])
        plsc.subcore_barrier()
        pltpu.sync_copy(sm_dinv.at[pl.ds(0, _NN)], dinv_v)

        def nissue(i, slot):
            b = wid * 10000 + i * nm_ch
            pltpu.async_copy(src_hbm.at[pl.ds(b, nm_ch)],
                             ibs[slot].at[pl.ds(0, nm_ch)], sems[slot])
            pltpu.async_copy(dst_hbm.at[pl.ds(b, nm_ch)],
                             ids[slot].at[pl.ds(0, nm_ch)], sems[slot])
            pltpu.async_copy(ew_hbm.at[pl.ds(b, nm_ch)],
                             fbs[slot].at[pl.ds(0, nm_ch)], sems[slot])

        def ndrain(slot):
            for hbm, buf in ((src_hbm, ibs[slot]), (dst_hbm, ids[slot]),
                             (ew_hbm, fbs[slot])):
                pltpu.make_async_copy(hbm.at[pl.ds(0, nm_ch)],
                                      buf.at[pl.ds(0, nm_ch)],
                                      sems[slot]).wait()

        nissue(0, 0)
        for i in range(nm_nch):
            slot = i % 2
            if i + 1 < nm_nch:
                nissue(i + 1, (i + 1) % 2)
            ndrain(slot)

            def ngrp(j, slot=slot):
                s16 = ibs[slot][pl.ds(j * 16, 16)]
                d16 = ids[slot][pl.ds(j * 16, 16)]
                w16 = fbs[slot][pl.ds(j * 16, 16)]
                nv = (plsc.load_gather(dinv_v, [s16]) * w16 *
                      plsc.load_gather(dinv_v, [d16]))
                nb[pl.ds(j * 16, 16)] = nv
            plsc.parallel_loop(0, nm_ch // 16, 1, unroll=10)(ngrp)
            pltpu.sync_copy(nb, norm_hbm.at[pl.ds(wid * 10000 + i * nm_ch,
                                                  nm_ch)])
        plsc.subcore_barrier()

        for j in range(FP):
            pltpu.sync_copy(
                xwT_hbm.at[pl.ds((fg * FP + j) * _NN, _NN)], tbl[j])

        def aissue(i, slot):
            b = es * edges + i * ch
            pltpu.async_copy(src_hbm.at[pl.ds(b, ch)], ibs[slot], sems[slot])
            pltpu.async_copy(dst_hbm.at[pl.ds(b, ch)], ids[slot], sems[slot])
            pltpu.async_copy(norm_hbm.at[pl.ds(b, ch)], fbs[slot], sems[slot])

        def adrain(slot):
            pltpu.make_async_copy(src_hbm.at[pl.ds(0, ch)], ibs[slot],
                                  sems[slot]).wait()
            pltpu.make_async_copy(dst_hbm.at[pl.ds(0, ch)], ids[slot],
                                  sems[slot]).wait()
            pltpu.make_async_copy(norm_hbm.at[pl.ds(0, ch)], fbs[slot],
                                  sems[slot]).wait()

        aissue(0, 0)
        aissue(1, 1)

        def pair_body(kk, _):
            i = kk * 2
            for slot in (0, 1):
                adrain(slot)

                def grp(g, slot=slot):
                    off = g * 16
                    s16 = ibs[slot][pl.ds(off, 16)]
                    d16 = ids[slot][pl.ds(off, 16)]
                    n16 = fbs[slot][pl.ds(off, 16)]
                    for j in range(FP):
                        v = plsc.load_gather(tbl[j], [s16]) * n16
                        plsc.addupdate_scatter(acc[j], [d16], v)
                plsc.parallel_loop(0, ch // 16, 1, unroll=unroll)(grp)

                @pl.when(i + 2 + slot < nch)
                def _(slot=slot):
                    aissue(i + 2 + slot, slot)
            return 0
        lax.fori_loop(0, nch // 2, pair_body, 0)

        for j in range(FP):
            pltpu.sync_copy(
                acc[j],
                out_hbm.at[pl.ds((es * F + fg * FP + j) * _NN, _NN)])

    return k(xw1T_flat, src, dst, ew)


def _make_agg(F, n_fg, n_es, ch, unroll=10):
    FP = F // n_fg
    edges = _E // n_es
    nch = edges // ch
    assert ch % (16 * unroll) == 0 and edges % ch == 0

    @functools.partial(
        pl.kernel,
        out_type=jax.ShapeDtypeStruct((n_es * F * _NN,), jnp.float32),
        mesh=_sc_mesh(),
        scratch_types=(
            [pltpu.VMEM((_NN,), jnp.float32) for _ in range(FP)] +
            [pltpu.VMEM((_NN,), jnp.float32) for _ in range(FP)] +
            [pltpu.VMEM((ch,), jnp.int32) for _ in range(2)] +
            [pltpu.VMEM((ch,), jnp.int32) for _ in range(2)] +
            [pltpu.VMEM((ch,), jnp.float32) for _ in range(2)] +
            [pltpu.SemaphoreType.DMA, pltpu.SemaphoreType.DMA]
        ),
        compiler_params=_SC_PARAMS,
    )
    def k(xwT_hbm, src_hbm, dst_hbm, norm_hbm, out_hbm, *scratch):
        tbl = scratch[:FP]
        acc = scratch[FP:2 * FP]
        ibs = scratch[2 * FP:2 * FP + 2]
        ids = scratch[2 * FP + 2:2 * FP + 4]
        fbs = scratch[2 * FP + 4:2 * FP + 6]
        sems = scratch[2 * FP + 6:]
        cid = lax.axis_index("c")
        sid = lax.axis_index("s")
        wid = cid * 16 + sid
        fg = wid % n_fg
        es = wid // n_fg
        base = es * edges

        def issue(i, slot):
            b = base + i * ch
            pltpu.async_copy(src_hbm.at[pl.ds(b, ch)], ibs[slot], sems[slot])
            pltpu.async_copy(dst_hbm.at[pl.ds(b, ch)], ids[slot], sems[slot])
            pltpu.async_copy(norm_hbm.at[pl.ds(b, ch)], fbs[slot], sems[slot])

        def drain(slot):
            pltpu.make_async_copy(src_hbm.at[pl.ds(0, ch)], ibs[slot],
                                  sems[slot]).wait()
            pltpu.make_async_copy(dst_hbm.at[pl.ds(0, ch)], ids[slot],
                                  sems[slot]).wait()
            pltpu.make_async_copy(norm_hbm.at[pl.ds(0, ch)], fbs[slot],
                                  sems[slot]).wait()

        def compute(slot):
            ib1, ib2, fb = ibs[slot], ids[slot], fbs[slot]

            def grp(g):
                off = g * 16
                s16 = ib1[pl.ds(off, 16)]
                d16 = ib2[pl.ds(off, 16)]
                n16 = fb[pl.ds(off, 16)]
                for j in range(FP):
                    v = plsc.load_gather(tbl[j], [s16]) * n16
                    plsc.addupdate_scatter(acc[j], [d16], v)
            plsc.parallel_loop(0, ch // 16, 1, unroll=unroll)(grp)

        issue(0, 0)
        issue(1, 1)

        for j in range(FP):
            pltpu.sync_copy(
                xwT_hbm.at[pl.ds((fg * FP + j) * _NN, _NN)], tbl[j])

        def zero_body(i):
            z = jnp.zeros((16,), jnp.float32)
            for j in range(FP):
                acc[j][pl.ds(i * 16, 16)] = z
        plsc.parallel_loop(0, _NN // 16, 1, unroll=5)(zero_body)

        def pair_body(kk, _):
            i = kk * 2
            for slot in (0, 1):
                drain(slot)
                compute(slot)

                @pl.when(i + 2 + slot < nch)
                def _(slot=slot):
                    issue(i + 2 + slot, slot)
            return 0
        lax.fori_loop(0, nch // 2, pair_body, 0)

        for j in range(FP):
            pltpu.sync_copy(
                acc[j],
                out_hbm.at[pl.ds((es * F + fg * FP + j) * _NN, _NN)])

    return k


def _mid_body(p_ref, xw_ref, sw_ref, b1_ref, g1_ref, be1_ref, w2_ref, o_ref):
    h = jnp.sum(p_ref[...], axis=0) + sw_ref[...] * xw_ref[...] + b1_ref[...]
    h = jnp.maximum(h, 0.0)
    mu = jnp.mean(h, axis=0, keepdims=True)
    var = jnp.mean((h - mu) ** 2, axis=0, keepdims=True)
    hn = (h - mu) * lax.rsqrt(var + 1e-5) * g1_ref[...] + be1_ref[...]
    o_ref[...] = lax.dot_general(w2_ref[...], hn, (((1,), (0,)), ((), ())),
                                 preferred_element_type=jnp.float32)


def _mid(parts, xw1T, selfw, b1, g1, be1, W2):
    return pl.pallas_call(
        _mid_body,
        out_shape=jax.ShapeDtypeStruct((32, _NN), jnp.float32),
    )(parts, xw1T, selfw[:_NN].reshape(1, _NN), b1.reshape(_H, 1),
      g1.reshape(_H, 1), be1.reshape(_H, 1), W2)


def _final_body(p_ref, xw_ref, sw_ref, b2_ref, g2_ref, be2_ref, bt_ref,
                wfc_ref, bfc_ref, x3_ref, x4_ref):
    h = jnp.sum(p_ref[...], axis=0) + sw_ref[...] * xw_ref[...] + b2_ref[...]
    h = jnp.maximum(h, 0.0)
    mu = jnp.mean(h, axis=0, keepdims=True)
    var = jnp.mean((h - mu) ** 2, axis=0, keepdims=True)
    x3 = (h - mu) * lax.rsqrt(var + 1e-5) * g2_ref[...] + be2_ref[...]
    x3_ref[...] = x3

    bb = bt_ref[...]
    gid = lax.broadcasted_iota(jnp.int32, (_G, _NN), 0)
    onehot = jnp.where(bb == gid, 1.0, 0.0)
    aug = jnp.concatenate([x3, jnp.ones((1, _NN), jnp.float32)], axis=0)
    a = lax.dot_general(onehot, aug, (((1,), (1,)), ((), ())),
                        preferred_element_type=jnp.float32)
    cnt = jnp.maximum(a[:, 32:33], 1.0)
    mean = a[:, :32] / cnt
    x4_ref[...] = lax.dot_general(
        mean, wfc_ref[...], (((1,), (1,)), ((), ())),
        preferred_element_type=jnp.float32) + bfc_ref[...]


def _final(parts, xw2T, selfw, b2, g2, be2, batch, Wfc, bfc):
    return pl.pallas_call(
        _final_body,
        out_shape=[
            jax.ShapeDtypeStruct((32, _NN), jnp.float32),
            jax.ShapeDtypeStruct((_G, _OUT), jnp.float32),
        ],
    )(parts, xw2T, selfw[:_NN].reshape(1, _NN), b2.reshape(32, 1),
      g2.reshape(32, 1), be2.reshape(32, 1),
      batch.reshape(1, _NN), Wfc, bfc.reshape(1, _OUT))


_make_agg = functools.lru_cache(maxsize=None)(_make_agg)


def kernel(x, edge_index, edge_attr, batch, W_att, b_att, W1, b1, g1, be1,
           W2, b2, g2, be2, Wfc, bfc):
    src, dst = edge_index[0], edge_index[1]
    xw1T = _attention(x, W_att, b_att, W1)
    selfw, norm, p1f = _norm_agg16(xw1T.reshape(-1), src, dst, edge_attr)
    p1 = p1f.reshape(8, _H, _NN)
    xw2T = _mid(p1, xw1T, selfw, b1, g1, be1, W2)
    p2 = _make_agg(32, n_fg=8, n_es=4, ch=4000)(
        xw2T.reshape(-1), src, dst, norm).reshape(4, 32, _NN)
    x3T, x4 = _final(p2, xw2T, selfw, b2, g2, be2, batch, Wfc, bfc)
    return (x3T.T, x4)
